# Initial kernel scaffold; baseline (speedup 1.0000x reference)
import functools

import jax
import jax.numpy as jnp
from jax import lax
from jax.experimental import pallas as pl
from jax.experimental.pallas import tpu as pltpu
from jax.experimental.pallas import tpu_sc as plsc

N = 10000
E = 320000
FEAT = 128
HEADS = 8
HEAD_DIM = 16
INV_SCALING = 4.0

C = 128
NCHUNKS = E // C
NW = 32
CHUNKS_PER_W = NCHUNKS // NW
CHUNKS_REM = NCHUNKS % NW
ROWS_PER_TILE = N // 16


def _qkv_body(x_ref, wq_ref, wk_ref, wv_ref, q_ref, k_ref, v_ref):
    xb = x_ref[...]
    dn = (((1,), (1,)), ((), ()))
    q_ref[...] = lax.dot_general(xb, wq_ref[...], dn,
                                 preferred_element_type=jnp.float32)
    k_ref[...] = lax.dot_general(xb, wk_ref[...], dn,
                                 preferred_element_type=jnp.float32)
    v_ref[...] = lax.dot_general(xb, wv_ref[...], dn,
                                 preferred_element_type=jnp.float32)


def _qkv(x, Wq, Wk, Wv):
    bn = 2500
    out = jax.ShapeDtypeStruct((N, FEAT), jnp.float32)
    return pl.pallas_call(
        _qkv_body,
        grid=(N // bn,),
        in_specs=[
            pl.BlockSpec((bn, FEAT), lambda i: (i, 0)),
            pl.BlockSpec((FEAT, FEAT), lambda i: (0, 0)),
            pl.BlockSpec((FEAT, FEAT), lambda i: (0, 0)),
            pl.BlockSpec((FEAT, FEAT), lambda i: (0, 0)),
        ],
        out_specs=[
            pl.BlockSpec((bn, FEAT), lambda i: (i, 0)),
            pl.BlockSpec((bn, FEAT), lambda i: (i, 0)),
            pl.BlockSpec((bn, FEAT), lambda i: (i, 0)),
        ],
        out_shape=[out, out, out],
    )(x, Wq, Wk, Wv)


def _edge_body(ea_ref, w2_ref, eb_ref, gt_ref):
    t = lax.dot_general(ea_ref[...], w2_ref[...], (((1,), (1,)), ((), ())),
                        preferred_element_type=jnp.float32)
    bias = t[:, :HEADS] * INV_SCALING
    gate = jax.nn.sigmoid(t[:, HEADS:])
    z = jnp.zeros_like(bias)
    eb_ref[...] = jnp.concatenate([bias, z], axis=1)
    gt_ref[...] = jnp.concatenate([gate, z], axis=1)


def _edge_feats(edge_attr, W2):
    be = 8000
    out = jax.ShapeDtypeStruct((E, 16), jnp.float32)
    return pl.pallas_call(
        _edge_body,
        grid=(E // be,),
        in_specs=[
            pl.BlockSpec((be, FEAT), lambda i: (i, 0)),
            pl.BlockSpec((16, FEAT), lambda i: (0, 0)),
        ],
        out_specs=[
            pl.BlockSpec((be, 16), lambda i: (i, 0)),
            pl.BlockSpec((be, 16), lambda i: (i, 0)),
        ],
        out_shape=[out, out],
    )(edge_attr, W2)


def _den_sum_body(d0_ref, d1_ref, den_ref):
    den_ref[...] = d0_ref[...] + d1_ref[...]


def _den_sum(d0, d1):
    return pl.pallas_call(
        _den_sum_body,
        out_shape=jax.ShapeDtypeStruct((N, 16), jnp.float32),
    )(d0, d1)


def _final_body(o0_ref, o1_ref, wn_ref, out_ref):
    agg = o0_ref[...] + o1_ref[...]
    out_ref[...] = lax.dot_general(agg, wn_ref[...], (((1,), (1,)), ((), ())),
                                   preferred_element_type=jnp.float32)


def _final(o0, o1, Wnode):
    bn = 2500
    return pl.pallas_call(
        _final_body,
        grid=(N // bn,),
        in_specs=[
            pl.BlockSpec((bn, FEAT), lambda i: (i, 0)),
            pl.BlockSpec((bn, FEAT), lambda i: (i, 0)),
            pl.BlockSpec((FEAT, FEAT), lambda i: (0, 0)),
        ],
        out_specs=pl.BlockSpec((bn, FEAT), lambda i: (i, 0)),
        out_shape=jax.ShapeDtypeStruct((N, FEAT), jnp.float32),
    )(o0, o1, Wnode)


def _mesh():
    return plsc.VectorSubcoreMesh(core_axis_name="c", subcore_axis_name="s")


def _pass_a(q, k, src, dst, eb, gt):
    f32 = jnp.float32

    @functools.partial(
        pl.kernel,
        out_type=[
            jax.ShapeDtypeStruct((E, 16), f32),
            jax.ShapeDtypeStruct((N, 16), f32),
            jax.ShapeDtypeStruct((N, 16), f32),
        ],
        mesh=_mesh(),
        scratch_types=[
            pltpu.VMEM((C,), jnp.int32),
            pltpu.VMEM((C,), jnp.int32),
            pltpu.VMEM((C, FEAT), f32),
            pltpu.VMEM((C, FEAT), f32),
            pltpu.VMEM((C, 16), f32),
            pltpu.VMEM((C, 16), f32),
            pltpu.VMEM((C, 16), f32),
            pltpu.VMEM((C, 16), f32),
            pltpu.VMEM((16,), f32),
            pltpu.VMEM_SHARED((N, 16), f32),
        ],
    )
    def kern(q_hbm, k_hbm, src_hbm, dst_hbm, eb_hbm, gt_hbm,
             num_hbm, den0_hbm, den1_hbm,
             idx_s, idx_d, qs, kd, ebb, gtb, numb, exsb, scr, den_sh):
        cid = lax.axis_index("c")
        sid = lax.axis_index("s")
        wid = sid * 2 + cid

        scr[...] = jnp.zeros((16,), f32)

        @pl.loop(0, C)
        def _(i):
            exsb[i, :] = jnp.zeros((16,), f32)

        @pl.loop(0, 5)
        def _(j):
            pltpu.sync_copy(
                exsb.at[pl.ds(0, 125)],
                den_sh.at[pl.ds(sid * ROWS_PER_TILE + j * 125, 125)])

        plsc.subcore_barrier()

        ng = CHUNKS_PER_W + jnp.where(wid < CHUNKS_REM, 1, 0)

        def chunk_body(g, carry):
            chunk = wid + g * NW
            base = pl.multiple_of(chunk * C, C)
            pltpu.sync_copy(src_hbm.at[pl.ds(base, C)], idx_s)
            pltpu.sync_copy(dst_hbm.at[pl.ds(base, C)], idx_d)
            pltpu.sync_copy(q_hbm.at[idx_s], qs)
            pltpu.sync_copy(k_hbm.at[idx_d], kd)
            pltpu.sync_copy(eb_hbm.at[pl.ds(base, C)], ebb)
            pltpu.sync_copy(gt_hbm.at[pl.ds(base, C)], gtb)

            @pl.loop(0, C)
            def _(e):
                for h in range(HEADS):
                    qv = qs[e, pl.ds(h * 16, 16)]
                    kv = kd[e, pl.ds(h * 16, 16)]
                    scr[h] = jnp.sum(qv * kv)
                av = scr[...]
                t = jnp.clip(av, -5.0, 5.0) * INV_SCALING + ebb[e, :]
                ex = jnp.exp(t)
                exsb[e, :] = ex
                numb[e, :] = ex * gtb[e, :]

            pltpu.sync_copy(numb, num_hbm.at[pl.ds(base, C)])
            pltpu.sync_copy(exsb, den_sh.at[idx_d], add=True)
            return carry

        lax.fori_loop(0, ng, chunk_body, 0)

        plsc.subcore_barrier()

        row0 = sid * ROWS_PER_TILE

        @pl.when(cid == 0)
        def _():
            pltpu.sync_copy(den_sh.at[pl.ds(row0, ROWS_PER_TILE)],
                            den0_hbm.at[pl.ds(row0, ROWS_PER_TILE)])

        @pl.when(cid == 1)
        def _():
            pltpu.sync_copy(den_sh.at[pl.ds(row0, ROWS_PER_TILE)],
                            den1_hbm.at[pl.ds(row0, ROWS_PER_TILE)])

    return kern(q, k, src, dst, eb, gt)


def _pass_b(v, src, dst, num, den):
    f32 = jnp.float32

    @functools.partial(
        pl.kernel,
        out_type=[
            jax.ShapeDtypeStruct((N, FEAT), f32),
            jax.ShapeDtypeStruct((N, FEAT), f32),
        ],
        mesh=_mesh(),
        scratch_types=[
            pltpu.VMEM((C,), jnp.int32),
            pltpu.VMEM((C,), jnp.int32),
            pltpu.VMEM((C, 16), f32),
            pltpu.VMEM((C, 16), f32),
            pltpu.VMEM((C, FEAT), f32),
            pltpu.VMEM_SHARED((N, FEAT), f32),
        ],
    )
    def kern(v_hbm, src_hbm, dst_hbm, num_hbm, den_hbm,
             out0_hbm, out1_hbm,
             idx_s, idx_d, numb, denb, vs, out_sh):
        cid = lax.axis_index("c")
        sid = lax.axis_index("s")
        wid = sid * 2 + cid

        @pl.loop(0, C)
        def _(i):
            @pl.loop(0, FEAT, step=16)
            def _(j):
                vs[i, pl.ds(j, 16)] = jnp.zeros((16,), f32)

        @pl.loop(0, 5)
        def _(j):
            pltpu.sync_copy(
                vs.at[pl.ds(0, 125)],
                out_sh.at[pl.ds(sid * ROWS_PER_TILE + j * 125, 125)])

        plsc.subcore_barrier()

        ng = CHUNKS_PER_W + jnp.where(wid < CHUNKS_REM, 1, 0)

        def chunk_body(g, carry):
            chunk = wid + g * NW
            base = pl.multiple_of(chunk * C, C)
            pltpu.sync_copy(src_hbm.at[pl.ds(base, C)], idx_s)
            pltpu.sync_copy(dst_hbm.at[pl.ds(base, C)], idx_d)
            pltpu.sync_copy(num_hbm.at[pl.ds(base, C)], numb)
            pltpu.sync_copy(den_hbm.at[idx_d], denb)
            pltpu.sync_copy(v_hbm.at[idx_s], vs)

            @pl.loop(0, C)
            def _(e):
                sa = numb[e, :] / denb[e, :]
                for h in range(HEADS):
                    s = lax.squeeze(lax.slice(sa, (h,), (h + 1,)), (0,))
                    vrow = vs[e, pl.ds(h * 16, 16)]
                    vs[e, pl.ds(h * 16, 16)] = vrow * s

            pltpu.sync_copy(vs, out_sh.at[idx_d], add=True)
            return carry

        lax.fori_loop(0, ng, chunk_body, 0)

        plsc.subcore_barrier()

        row0 = sid * ROWS_PER_TILE

        @pl.when(cid == 0)
        def _():
            pltpu.sync_copy(out_sh.at[pl.ds(row0, ROWS_PER_TILE)],
                            out0_hbm.at[pl.ds(row0, ROWS_PER_TILE)])

        @pl.when(cid == 1)
        def _():
            pltpu.sync_copy(out_sh.at[pl.ds(row0, ROWS_PER_TILE)],
                            out1_hbm.at[pl.ds(row0, ROWS_PER_TILE)])

    return kern(v, src, dst, num, den)


def kernel(x, edge_index, edge_attr, Wq, Wk, Wv, Wnode, Wedge, Wgate):
    src = edge_index[0]
    dst = edge_index[1]
    W2 = jnp.concatenate([Wedge, Wgate], axis=0)

    q, k, v = _qkv(x, Wq, Wk, Wv)
    eb, gt = _edge_feats(edge_attr, W2)
    num, den0, den1 = _pass_a(q, k, src, dst, eb, gt)
    den = _den_sum(den0, den1)
    o0, o1 = _pass_b(v, src, dst, num, den)
    return _final(o0, o1, Wnode)


# speedup vs baseline: 25.2782x; 25.2782x over previous
import dataclasses
import functools

import jax
import jax.numpy as jnp
from jax import lax
from jax.experimental import pallas as pl
from jax.experimental.pallas import tpu as pltpu
from jax.experimental.pallas import tpu_sc as plsc

N = 10000
E = 320000
FEAT = 128
HEADS = 8
HEAD_DIM = 16
INV_SCALING = 4.0

C = 128
NCHUNKS = E // C
NW = 32
CHUNKS_PER_W = NCHUNKS // NW
CHUNKS_REM = NCHUNKS % NW
N_PAD = 10240
ROWS_PER_TILE = N_PAD // 16
D8 = N_PAD // 8
D8T = D8 // 16
E8 = E // 8
CR = C // 8


def _qkv_body(x_ref, wq_ref, wk_ref, wv_ref, q_ref, k_ref, v_ref):
    xb = x_ref[...]
    dn = (((1,), (1,)), ((), ()))
    q_ref[...] = lax.dot_general(xb, wq_ref[...], dn,
                                 preferred_element_type=jnp.float32)
    k_ref[...] = lax.dot_general(xb, wk_ref[...], dn,
                                 preferred_element_type=jnp.float32)
    v_ref[...] = lax.dot_general(xb, wv_ref[...], dn,
                                 preferred_element_type=jnp.float32)


def _qkv(x, Wq, Wk, Wv):
    bn = 2000
    out = jax.ShapeDtypeStruct((N, FEAT), jnp.float32)
    return pl.pallas_call(
        _qkv_body,
        grid=(N // bn,),
        in_specs=[
            pl.BlockSpec((bn, FEAT), lambda i: (i, 0)),
            pl.BlockSpec((FEAT, FEAT), lambda i: (0, 0)),
            pl.BlockSpec((FEAT, FEAT), lambda i: (0, 0)),
            pl.BlockSpec((FEAT, FEAT), lambda i: (0, 0)),
        ],
        out_specs=[
            pl.BlockSpec((bn, FEAT), lambda i: (i, 0)),
            pl.BlockSpec((bn, FEAT), lambda i: (i, 0)),
            pl.BlockSpec((bn, FEAT), lambda i: (i, 0)),
        ],
        out_shape=[out, out, out],
    )(x, Wq, Wk, Wv)


def _edge_body(ea_ref, w2_ref, eb_ref, gt_ref):
    t = lax.dot_general(ea_ref[...], w2_ref[...], (((1,), (1,)), ((), ())),
                        preferred_element_type=jnp.float32)
    bias = t[:, :HEADS] * INV_SCALING
    gate = jax.nn.sigmoid(t[:, HEADS:])
    z = jnp.zeros_like(bias)
    eb_ref[...] = jnp.concatenate([bias, z], axis=1)
    gt_ref[...] = jnp.concatenate([gate, z], axis=1)


def _edge_feats(edge_attr, W2):
    be = 8000
    out = jax.ShapeDtypeStruct((E, 16), jnp.float32)
    return pl.pallas_call(
        _edge_body,
        grid=(E // be,),
        in_specs=[
            pl.BlockSpec((be, FEAT), lambda i: (i, 0)),
            pl.BlockSpec((16, FEAT), lambda i: (0, 0)),
        ],
        out_specs=[
            pl.BlockSpec((be, 16), lambda i: (i, 0)),
            pl.BlockSpec((be, 16), lambda i: (i, 0)),
        ],
        out_shape=[out, out],
    )(edge_attr, W2)


def _den_recip_body(d2_ref, den_ref):
    den_ref[...] = 1.0 / (d2_ref[:D8, :] + d2_ref[D8:, :])


def _den_recip(d2):
    return pl.pallas_call(
        _den_recip_body,
        out_shape=jax.ShapeDtypeStruct((D8, FEAT), jnp.float32),
    )(d2)


def _final_body(o0_ref, o1_ref, wn_ref, out_ref):
    agg = o0_ref[...] + o1_ref[...]
    out_ref[...] = lax.dot_general(agg, wn_ref[...], (((1,), (1,)), ((), ())),
                                   preferred_element_type=jnp.float32)


def _final(oo, Wnode):
    bn = N_PAD // 8
    nblk = N_PAD // bn
    return pl.pallas_call(
        _final_body,
        grid=(nblk,),
        in_specs=[
            pl.BlockSpec((bn, FEAT), lambda i: (i, 0)),
            pl.BlockSpec((bn, FEAT), lambda i, _n=nblk: (i + _n, 0)),
            pl.BlockSpec((FEAT, FEAT), lambda i: (0, 0)),
        ],
        out_specs=pl.BlockSpec((bn, FEAT), lambda i: (i, 0)),
        out_shape=jax.ShapeDtypeStruct((N_PAD, FEAT), jnp.float32),
    )(oo, oo, Wnode)


def _mesh():
    return plsc.VectorSubcoreMesh(core_axis_name="c", subcore_axis_name="s")


def _sc_params():
    cp = pltpu.CompilerParams()
    if "needs_layout_passes" in pltpu.CompilerParams.__dataclass_fields__:
        cp = dataclasses.replace(cp, needs_layout_passes=False)
    return cp


def _pass_a(q, k, src, dst, eb8, gt8):
    f32 = jnp.float32

    @functools.partial(
        pl.kernel,
        out_type=[
            jax.ShapeDtypeStruct((E8, FEAT), f32),
            jax.ShapeDtypeStruct((2 * D8, FEAT), f32),
        ],
        mesh=_mesh(),
        scratch_types=[
            pltpu.VMEM((C,), jnp.int32),
            pltpu.VMEM((C,), jnp.int32),
            pltpu.VMEM((C,), jnp.int32),
            pltpu.VMEM((C, FEAT), f32),
            pltpu.VMEM((C, FEAT), f32),
            pltpu.VMEM((CR, FEAT), f32),
            pltpu.VMEM((CR, FEAT), f32),
            pltpu.VMEM((CR, FEAT), f32),
            pltpu.VMEM((C, FEAT), f32),
            pltpu.VMEM_SHARED((D8, FEAT), f32),
        ],
        compiler_params=_sc_params(),
    )
    def kern(q_hbm, k_hbm, src_hbm, dst_hbm, eb_hbm, gt_hbm,
             num_hbm, den2_hbm,
             idx_s, idx_d, idx_r, qs, kd, ebb, gtb, numb, exs, den_sh):
        cid = lax.axis_index("c")
        sid = lax.axis_index("s")
        wid = sid * 2 + cid
        lane = lax.iota(jnp.int32, 16)
        lane_masks = [lane == h for h in range(HEADS)]
        zero16 = jnp.zeros((16,), f32)
        lane_i = [jnp.full((16, 1), i, jnp.int32) for i in range(16)]
        gdn = lax.GatherDimensionNumbers(
            offset_dims=(), collapsed_slice_dims=(0,), start_index_map=(0,))

        def _bcast(vec, idx):
            return lax.gather(vec, idx, gdn, (1,),
                              mode=lax.GatherScatterMode.PROMISE_IN_BOUNDS)

        @pl.loop(0, C)
        def _(e):
            @pl.loop(0, FEAT, step=16)
            def _(j):
                exs[e, pl.ds(j, 16)] = zero16

        pltpu.sync_copy(exs.at[pl.ds(0, D8T)],
                        den_sh.at[pl.ds(sid * D8T, D8T)])

        plsc.subcore_barrier()

        ng = CHUNKS_PER_W + jnp.where(wid < CHUNKS_REM, 1, 0)

        def chunk_body(g, carry):
            chunk = wid + g * NW
            base = pl.multiple_of(chunk * C, C)
            base8 = pl.multiple_of(chunk * CR, CR)
            pltpu.sync_copy(src_hbm.at[pl.ds(base, C)], idx_s)
            pltpu.sync_copy(dst_hbm.at[pl.ds(base, C)], idx_d)
            pltpu.sync_copy(q_hbm.at[idx_s], qs)
            pltpu.sync_copy(k_hbm.at[idx_d], kd)
            pltpu.sync_copy(eb_hbm.at[pl.ds(base8, CR)], ebb)
            pltpu.sync_copy(gt_hbm.at[pl.ds(base8, CR)], gtb)

            @pl.loop(0, C, step=16)
            def _(j):
                idx_r[pl.ds(j, 16)] = jnp.right_shift(idx_d[pl.ds(j, 16)], 3)

            @pl.loop(0, C, step=16)
            def _(j0):
                r0 = jnp.right_shift(j0, 3)
                dj = idx_d[pl.ds(j0, 16)]
                for i in range(16):
                    e = j0 + i
                    r = r0 + (i >> 3)
                    ce = (i & 7) * 16
                    av = jnp.zeros((16,), f32)
                    for h in range(HEADS):
                        qv = qs[e, pl.ds(h * 16, 16)]
                        kv = kd[e, pl.ds(h * 16, 16)]
                        s = jnp.sum(qv * kv)
                        av = jnp.where(lane_masks[h], s, av)
                    ebv = ebb[r, pl.ds(ce, 16)]
                    gtv = gtb[r, pl.ds(ce, 16)]
                    t = jnp.clip(av, -5.0, 5.0) * INV_SCALING + ebv
                    ex = jnp.exp(t)
                    dv8 = jnp.bitwise_and(_bcast(dj, lane_i[i]), 7)
                    for sl in range(8):
                        exs[e, pl.ds(sl * 16, 16)] = jnp.where(
                            dv8 == sl, ex, zero16)
                    numb[r, pl.ds(ce, 16)] = ex * gtv

            pltpu.sync_copy(numb, num_hbm.at[pl.ds(base8, CR)])
            pltpu.sync_copy(exs, den_sh.at[idx_r], add=True)
            return carry

        lax.fori_loop(0, ng, chunk_body, 0)

        plsc.subcore_barrier()

        row0 = sid * D8T
        pltpu.sync_copy(den_sh.at[pl.ds(row0, D8T)], exs.at[pl.ds(0, D8T)])
        pltpu.sync_copy(exs.at[pl.ds(0, D8T)],
                        den2_hbm.at[pl.ds(cid * D8 + row0, D8T)])

    return kern(q, k, src, dst, eb8, gt8)


def _pass_b(v, src, dst, num8, rden):
    f32 = jnp.float32

    @functools.partial(
        pl.kernel,
        out_type=jax.ShapeDtypeStruct((2 * N_PAD, FEAT), f32),
        mesh=_mesh(),
        scratch_types=[
            pltpu.VMEM((C,), jnp.int32),
            pltpu.VMEM((C,), jnp.int32),
            pltpu.VMEM((C,), jnp.int32),
            pltpu.VMEM((CR, FEAT), f32),
            pltpu.VMEM((C, FEAT), f32),
            pltpu.VMEM((C, FEAT), f32),
            pltpu.VMEM_SHARED((N_PAD, FEAT), f32),
        ],
        compiler_params=_sc_params(),
    )
    def kern(v_hbm, src_hbm, dst_hbm, num_hbm, rden_hbm,
             oo_hbm,
             idx_s, idx_d, idx_r, numb, gden, vs, out_sh):
        cid = lax.axis_index("c")
        sid = lax.axis_index("s")
        wid = sid * 2 + cid
        row0 = sid * ROWS_PER_TILE
        zero16 = jnp.zeros((16,), f32)
        lane = lax.iota(jnp.int32, 16)
        head_idx = [jnp.full((16, 1), h, jnp.int32) for h in range(HEADS)]
        lane_i = [jnp.full((16, 1), i, jnp.int32) for i in range(16)]
        gdn = lax.GatherDimensionNumbers(
            offset_dims=(), collapsed_slice_dims=(0,), start_index_map=(0,))

        def _bcast(vec, idx):
            return lax.gather(vec, idx, gdn, (1,),
                              mode=lax.GatherScatterMode.PROMISE_IN_BOUNDS)

        @pl.loop(0, C)
        def _(i):
            @pl.loop(0, FEAT, step=16)
            def _(j):
                vs[i, pl.ds(j, 16)] = zero16

        @pl.loop(0, 5)
        def _(j):
            pltpu.sync_copy(vs, out_sh.at[pl.ds(row0 + j * C, C)])

        plsc.subcore_barrier()

        ng = CHUNKS_PER_W + jnp.where(wid < CHUNKS_REM, 1, 0)

        def chunk_body(g, carry):
            chunk = wid + g * NW
            base = pl.multiple_of(chunk * C, C)
            base8 = pl.multiple_of(chunk * CR, CR)
            pltpu.sync_copy(src_hbm.at[pl.ds(base, C)], idx_s)
            pltpu.sync_copy(dst_hbm.at[pl.ds(base, C)], idx_d)
            pltpu.sync_copy(num_hbm.at[pl.ds(base8, CR)], numb)

            @pl.loop(0, C, step=16)
            def _(j):
                idx_r[pl.ds(j, 16)] = jnp.right_shift(idx_d[pl.ds(j, 16)], 3)

            pltpu.sync_copy(rden_hbm.at[idx_r], gden)
            pltpu.sync_copy(v_hbm.at[idx_s], vs)

            @pl.loop(0, C, step=16)
            def _(j0):
                r0 = jnp.right_shift(j0, 3)
                dj = idx_d[pl.ds(j0, 16)]
                for i in range(16):
                    e = j0 + i
                    r = r0 + (i >> 3)
                    ce = (i & 7) * 16
                    dv = _bcast(dj, lane_i[i])
                    cbv = jnp.bitwise_and(dv, 7) * 16 + lane
                    evec = jnp.full((16,), e, jnp.int32)
                    gv = plsc.load_gather(gden, [evec, cbv])
                    sa = numb[r, pl.ds(ce, 16)] * gv
                    for h in range(HEADS):
                        sb = _bcast(sa, head_idx[h])
                        vrow = vs[e, pl.ds(h * 16, 16)]
                        vs[e, pl.ds(h * 16, 16)] = vrow * sb

            pltpu.sync_copy(vs, out_sh.at[idx_d], add=True)
            return carry

        lax.fori_loop(0, ng, chunk_body, 0)

        plsc.subcore_barrier()

        obase = cid * N_PAD + row0

        @pl.loop(0, 5)
        def _(j):
            pltpu.sync_copy(out_sh.at[pl.ds(row0 + j * C, C)], vs)
            pltpu.sync_copy(vs, oo_hbm.at[pl.ds(obase + j * C, C)])

    return kern(v, src, dst, num8, rden)


def kernel(x, edge_index, edge_attr, Wq, Wk, Wv, Wnode, Wedge, Wgate):
    src = edge_index[0]
    dst = edge_index[1]
    W2 = jnp.concatenate([Wedge, Wgate], axis=0)

    q, k, v = _qkv(x, Wq, Wk, Wv)
    eb, gt = _edge_feats(edge_attr, W2)
    eb8 = eb.reshape(E8, FEAT)
    gt8 = gt.reshape(E8, FEAT)
    num8, den2 = _pass_a(q, k, src, dst, eb8, gt8)
    rden = _den_recip(den2)
    oo = _pass_b(v, src, dst, num8, rden)
    return _final(oo, Wnode)[:N]
